# baseline (speedup 1.0000x reference)
import functools
import jax
import jax.numpy as jnp
from jax import lax
from jax.experimental import pallas as pl
from jax.experimental.pallas import tpu as pltpu
from jax.experimental.pallas import tpu_sc as plsc

B, N, H, W = 8, 256, 512, 512
MAX_ITERS = 20
NC, NS, L = 2, 16, 16
NW = NC * NS
PTS = B * N
PPW = PTS // NW
WPB = N // PPW
GRPS = PPW // L
NBR = 9
OFFS = [(dy, dx) for dy in (-1, 0, 1) for dx in (-1, 0, 1)]
HPTS = PPW // 2
HFLAT = NBR * HPTS
HG = GRPS // 2


MROWS = H * 4
DROWS = MROWS + 8
DSTR = DROWS * 128


def _smooth_body(d_ref, p_ref, o_ref):
    a = d_ref[0, 0]
    zr = jnp.zeros((1, W), jnp.float32)
    rs = a + jnp.concatenate([a[1:], zr], 0) + jnp.concatenate([zr, a[:-1]], 0)
    zc = jnp.zeros((H, 1), jnp.float32)
    cs = rs + jnp.concatenate([rs[:, 1:], zc], 1) + jnp.concatenate([zc, rs[:, :-1]], 1)
    o_ref[0, :MROWS] = (cs * jnp.float32(1.0 / 9.0)).reshape(MROWS, 128)
    ybits = jax.lax.bitcast_convert_type(p_ref[0, :, 0], jnp.float32)
    xbits = jax.lax.bitcast_convert_type(p_ref[0, :, 1], jnp.float32)
    o_ref[0, MROWS:MROWS + 2] = ybits.reshape(2, 128)
    o_ref[0, MROWS + 2:MROWS + 4] = xbits.reshape(2, 128)
    o_ref[0, MROWS + 4:] = jnp.zeros((4, 128), jnp.float32)


_smooth_call = pl.pallas_call(
    _smooth_body,
    out_shape=jax.ShapeDtypeStruct((B, DROWS, 128), jnp.float32),
    grid=(B,),
    in_specs=[pl.BlockSpec((1, 1, H, W), lambda b: (b, 0, 0, 0)),
              pl.BlockSpec((1, N, 2), lambda b: (b, 0, 0))],
    out_specs=pl.BlockSpec((1, DROWS, 128), lambda b: (b, 0, 0)),
)


def _climb_body(d_hbm, out_hbm,
                pin, ycur, xcur, idxA, valsA, idxB, valsB, obuf, shared,
                semA, semB):
    cid = lax.axis_index("c")
    sid = lax.axis_index("s")
    wid = cid * NS + sid
    base_pt = wid * PPW
    bat = cid * WPB + sid // WPB
    quarter = sid % WPB
    boff = (sid // WPB) * (H * W)

    SEG = H * W // WPB
    pltpu.sync_copy(d_hbm.at[pl.ds(bat * DSTR + quarter * SEG, SEG)],
                    shared.at[pl.ds(sid * SEG, SEG)])

    lane16 = lax.iota(jnp.int32, L)

    pltpu.sync_copy(d_hbm.at[pl.ds(bat * DSTR + H * W + quarter * PPW, PPW)],
                    pin.at[pl.ds(0, PPW)])
    pltpu.sync_copy(d_hbm.at[pl.ds(bat * DSTR + H * W + N + quarter * PPW, PPW)],
                    pin.at[pl.ds(PPW, PPW)])
    plsc.subcore_barrier()
    for g in range(GRPS):
        sl = pl.ds(g * L, L)
        ycur[sl] = plsc.bitcast(pin[pl.ds(g * L, L)], jnp.int32)
        xcur[sl] = plsc.bitcast(pin[pl.ds(PPW + g * L, L)], jnp.int32)

    halves = ((idxA, valsA, semA, 0), (idxB, valsB, semB, 1))

    def compute_idx(idx_ref, h):
        for g in range(HG):
            g_abs = 2 * h + g
            sl = pl.ds(g_abs * L, L)
            yv = ycur[sl]
            xv = xcur[sl]
            cyd = {dy: (jnp.clip(yv + dy, 0, H - 1) << 9) + boff for dy in (-1, 0, 1)}
            cxd = {dx: jnp.clip(xv + dx, 0, W - 1) for dx in (-1, 0, 1)}
            for k, (dy, dx) in enumerate(OFFS):
                idx_ref[pl.ds(k * HPTS + g * L, L)] = cyd[dy] + cxd[dx]

    def fire(idx_ref, vals_ref, sem):
        return [pltpu.async_copy(shared.at[idx_ref.at[pl.ds(o, 96)]],
                                 vals_ref.at[pl.ds(o, 96)], sem)
                for o in (0, 96, 192)]

    def drain(copies):
        for c in copies:
            c.wait()

    def advance(vals_ref, h):
        for g in range(HG):
            g_abs = 2 * h + g
            sl = pl.ds(g_abs * L, L)
            yv = ycur[sl]
            xv = xcur[sl]
            bv = bdy = bdx = None
            for k, (dy, dx) in enumerate(OFFS):
                val = vals_ref[pl.ds(k * HPTS + g * L, L)]
                if k == 0:
                    bv = val
                    bdy = jnp.full((L,), dy, jnp.int32)
                    bdx = jnp.full((L,), dx, jnp.int32)
                else:
                    m = val > bv
                    bv = jnp.where(m, val, bv)
                    bdy = jnp.where(m, jnp.int32(dy), bdy)
                    bdx = jnp.where(m, jnp.int32(dx), bdx)
            ycur[sl] = jnp.clip(yv + bdy, 0, H - 1)
            xcur[sl] = jnp.clip(xv + bdx, 0, W - 1)

    compute_idx(idxA, 0)
    fire(idxA, valsA, semA)
    compute_idx(idxB, 1)
    fire(idxB, valsB, semB)

    def body2(_, carry):
        drain([pltpu.make_async_copy(shared.at[idxA.at[pl.ds(o, 96)]],
                                     valsA.at[pl.ds(o, 96)], semA)
               for o in (0, 96, 192)])
        advance(valsA, 0)
        compute_idx(idxA, 0)
        fire(idxA, valsA, semA)
        drain([pltpu.make_async_copy(shared.at[idxB.at[pl.ds(o, 96)]],
                                     valsB.at[pl.ds(o, 96)], semB)
               for o in (0, 96, 192)])
        advance(valsB, 1)
        compute_idx(idxB, 1)
        fire(idxB, valsB, semB)
        return carry

    lax.fori_loop(0, MAX_ITERS, body2, 0)

    drain([pltpu.make_async_copy(shared.at[idxA.at[pl.ds(o, 96)]],
                                 valsA.at[pl.ds(o, 96)], semA)
           for o in (0, 96, 192)])
    drain([pltpu.make_async_copy(shared.at[idxB.at[pl.ds(o, 96)]],
                                 valsB.at[pl.ds(o, 96)], semB)
           for o in (0, 96, 192)])

    for g in range(GRPS):
        sl = pl.ds(g * L, L)
        oi = (lane16 << 1) + (2 * g * L)
        plsc.store_scatter(obuf, [oi], ycur[sl].astype(jnp.float32))
        plsc.store_scatter(obuf, [oi + 1], xcur[sl].astype(jnp.float32))
    pltpu.sync_copy(obuf, out_hbm.at[pl.ds(base_pt * 2, 2 * PPW)])


@functools.cache
def _climb_call():
    return pl.kernel(
        _climb_body,
        out_type=jax.ShapeDtypeStruct((PTS * 2,), jnp.float32),
        mesh=plsc.VectorSubcoreMesh(core_axis_name="c", subcore_axis_name="s",
                                    num_cores=NC, num_subcores=NS),
        scratch_types=[
            pltpu.VMEM((2 * PPW,), jnp.float32),
            pltpu.VMEM((PPW,), jnp.int32),
            pltpu.VMEM((PPW,), jnp.int32),
            pltpu.VMEM((HFLAT,), jnp.int32),
            pltpu.VMEM((HFLAT,), jnp.float32),
            pltpu.VMEM((HFLAT,), jnp.int32),
            pltpu.VMEM((HFLAT,), jnp.float32),
            pltpu.VMEM((2 * PPW,), jnp.float32),
            pltpu.VMEM_SHARED((WPB * H * W,), jnp.float32),
            pltpu.SemaphoreType.DMA,
            pltpu.SemaphoreType.DMA,
        ],
        compiler_params=pltpu.CompilerParams(needs_layout_passes=False),
    )


def kernel(depth, points):
    d = _smooth_call(depth, points)
    out = _climb_call()(d.reshape(B * DSTR))
    return out.reshape(B, N, 2)


# speedup vs baseline: 1.0354x; 1.0354x over previous
import functools
import jax
import jax.numpy as jnp
from jax import lax
from jax.experimental import pallas as pl
from jax.experimental.pallas import tpu as pltpu
from jax.experimental.pallas import tpu_sc as plsc

B, N, H, W = 8, 256, 512, 512
MAX_ITERS = 20
NC, NS, L = 2, 16, 16
NW = NC * NS
PTS = B * N
PPW = PTS // NW
WPB = N // PPW
GRPS = PPW // L
NBR = 9
OFFS = [(dy, dx) for dy in (-1, 0, 1) for dx in (-1, 0, 1)]
HPTS = PPW // 2
HFLAT = NBR * HPTS
HG = GRPS // 2


MROWS = H * 4
DROWS = MROWS + 8
DSTR = DROWS * 128


SMB = 4


def _smooth_body(d_ref, p_ref, o_ref):
    for i in range(SMB):
        a = d_ref[i, 0]
        zr = jnp.zeros((1, W), jnp.float32)
        rs = a + jnp.concatenate([a[1:], zr], 0) + jnp.concatenate([zr, a[:-1]], 0)
        zc = jnp.zeros((H, 1), jnp.float32)
        cs = rs + jnp.concatenate([rs[:, 1:], zc], 1) + jnp.concatenate([zc, rs[:, :-1]], 1)
        o_ref[i, :MROWS] = (cs * jnp.float32(1.0 / 9.0)).reshape(MROWS, 128)
        ybits = jax.lax.bitcast_convert_type(p_ref[i, :, 0], jnp.float32)
        xbits = jax.lax.bitcast_convert_type(p_ref[i, :, 1], jnp.float32)
        o_ref[i, MROWS:MROWS + 2] = ybits.reshape(2, 128)
        o_ref[i, MROWS + 2:MROWS + 4] = xbits.reshape(2, 128)
        o_ref[i, MROWS + 4:] = jnp.zeros((4, 128), jnp.float32)


_smooth_call = pl.pallas_call(
    _smooth_body,
    out_shape=jax.ShapeDtypeStruct((B, DROWS, 128), jnp.float32),
    grid=(B // SMB,),
    in_specs=[pl.BlockSpec((SMB, 1, H, W), lambda b: (b, 0, 0, 0)),
              pl.BlockSpec((SMB, N, 2), lambda b: (b, 0, 0))],
    out_specs=pl.BlockSpec((SMB, DROWS, 128), lambda b: (b, 0, 0)),
)


def _climb_body(d_hbm, out_hbm,
                pin, ycur, xcur, idxA, valsA, idxB, valsB, obuf, shared,
                semA, semB):
    cid = lax.axis_index("c")
    sid = lax.axis_index("s")
    wid = cid * NS + sid
    base_pt = wid * PPW
    bat = cid * WPB + sid // WPB
    quarter = sid % WPB
    boff = (sid // WPB) * (H * W)

    SEG = H * W // WPB
    pltpu.sync_copy(d_hbm.at[pl.ds(bat * DSTR + quarter * SEG, SEG)],
                    shared.at[pl.ds(sid * SEG, SEG)])

    lane16 = lax.iota(jnp.int32, L)

    pltpu.sync_copy(d_hbm.at[pl.ds(bat * DSTR + H * W + quarter * PPW, PPW)],
                    pin.at[pl.ds(0, PPW)])
    pltpu.sync_copy(d_hbm.at[pl.ds(bat * DSTR + H * W + N + quarter * PPW, PPW)],
                    pin.at[pl.ds(PPW, PPW)])
    plsc.subcore_barrier()
    for g in range(GRPS):
        sl = pl.ds(g * L, L)
        ycur[sl] = plsc.bitcast(pin[pl.ds(g * L, L)], jnp.int32)
        xcur[sl] = plsc.bitcast(pin[pl.ds(PPW + g * L, L)], jnp.int32)

    halves = ((idxA, valsA, semA, 0), (idxB, valsB, semB, 1))

    def compute_idx(idx_ref, h):
        for g in range(HG):
            g_abs = 2 * h + g
            sl = pl.ds(g_abs * L, L)
            yv = ycur[sl]
            xv = xcur[sl]
            cyd = {dy: (jnp.clip(yv + dy, 0, H - 1) << 9) + boff for dy in (-1, 0, 1)}
            cxd = {dx: jnp.clip(xv + dx, 0, W - 1) for dx in (-1, 0, 1)}
            for k, (dy, dx) in enumerate(OFFS):
                idx_ref[pl.ds(k * HPTS + g * L, L)] = cyd[dy] + cxd[dx]

    def fire(idx_ref, vals_ref, sem):
        return [pltpu.async_copy(shared.at[idx_ref.at[pl.ds(o, 96)]],
                                 vals_ref.at[pl.ds(o, 96)], sem)
                for o in (0, 96, 192)]

    def drain(copies):
        for c in copies:
            c.wait()

    def advance(vals_ref, h):
        for g in range(HG):
            g_abs = 2 * h + g
            sl = pl.ds(g_abs * L, L)
            yv = ycur[sl]
            xv = xcur[sl]
            bv = bdy = bdx = None
            for k, (dy, dx) in enumerate(OFFS):
                val = vals_ref[pl.ds(k * HPTS + g * L, L)]
                if k == 0:
                    bv = val
                    bdy = jnp.full((L,), dy, jnp.int32)
                    bdx = jnp.full((L,), dx, jnp.int32)
                else:
                    m = val > bv
                    bv = jnp.where(m, val, bv)
                    bdy = jnp.where(m, jnp.int32(dy), bdy)
                    bdx = jnp.where(m, jnp.int32(dx), bdx)
            ycur[sl] = jnp.clip(yv + bdy, 0, H - 1)
            xcur[sl] = jnp.clip(xv + bdx, 0, W - 1)

    compute_idx(idxA, 0)
    fire(idxA, valsA, semA)
    compute_idx(idxB, 1)
    fire(idxB, valsB, semB)

    def body2(_, carry):
        drain([pltpu.make_async_copy(shared.at[idxA.at[pl.ds(o, 96)]],
                                     valsA.at[pl.ds(o, 96)], semA)
               for o in (0, 96, 192)])
        advance(valsA, 0)
        compute_idx(idxA, 0)
        fire(idxA, valsA, semA)
        drain([pltpu.make_async_copy(shared.at[idxB.at[pl.ds(o, 96)]],
                                     valsB.at[pl.ds(o, 96)], semB)
               for o in (0, 96, 192)])
        advance(valsB, 1)
        compute_idx(idxB, 1)
        fire(idxB, valsB, semB)
        return carry

    lax.fori_loop(0, MAX_ITERS, body2, 0)

    drain([pltpu.make_async_copy(shared.at[idxA.at[pl.ds(o, 96)]],
                                 valsA.at[pl.ds(o, 96)], semA)
           for o in (0, 96, 192)])
    drain([pltpu.make_async_copy(shared.at[idxB.at[pl.ds(o, 96)]],
                                 valsB.at[pl.ds(o, 96)], semB)
           for o in (0, 96, 192)])

    for g in range(GRPS):
        sl = pl.ds(g * L, L)
        oi = (lane16 << 1) + (2 * g * L)
        plsc.store_scatter(obuf, [oi], ycur[sl].astype(jnp.float32))
        plsc.store_scatter(obuf, [oi + 1], xcur[sl].astype(jnp.float32))
    pltpu.sync_copy(obuf, out_hbm.at[pl.ds(base_pt * 2, 2 * PPW)])


@functools.cache
def _climb_call():
    return pl.kernel(
        _climb_body,
        out_type=jax.ShapeDtypeStruct((PTS * 2,), jnp.float32),
        mesh=plsc.VectorSubcoreMesh(core_axis_name="c", subcore_axis_name="s",
                                    num_cores=NC, num_subcores=NS),
        scratch_types=[
            pltpu.VMEM((2 * PPW,), jnp.float32),
            pltpu.VMEM((PPW,), jnp.int32),
            pltpu.VMEM((PPW,), jnp.int32),
            pltpu.VMEM((HFLAT,), jnp.int32),
            pltpu.VMEM((HFLAT,), jnp.float32),
            pltpu.VMEM((HFLAT,), jnp.int32),
            pltpu.VMEM((HFLAT,), jnp.float32),
            pltpu.VMEM((2 * PPW,), jnp.float32),
            pltpu.VMEM_SHARED((WPB * H * W,), jnp.float32),
            pltpu.SemaphoreType.DMA,
            pltpu.SemaphoreType.DMA,
        ],
        compiler_params=pltpu.CompilerParams(needs_layout_passes=False),
    )


def kernel(depth, points):
    d = _smooth_call(depth, points)
    out = _climb_call()(d.reshape(B * DSTR))
    return out.reshape(B, N, 2)


# speedup vs baseline: 1.0383x; 1.0028x over previous
import functools
import jax
import jax.numpy as jnp
from jax import lax
from jax.experimental import pallas as pl
from jax.experimental.pallas import tpu as pltpu
from jax.experimental.pallas import tpu_sc as plsc

B, N, H, W = 8, 256, 512, 512
MAX_ITERS = 20
NC, NS, L = 2, 16, 16
NW = NC * NS
PTS = B * N
PPW = PTS // NW
WPB = N // PPW
GRPS = PPW // L
NBR = 9
OFFS = [(dy, dx) for dy in (-1, 0, 1) for dx in (-1, 0, 1)]
HPTS = PPW // 2
OFFS8 = [o for o in OFFS if o != (0, 0)]
HFLAT = 8 * HPTS
HG = GRPS // 2


MROWS = H * 4
DROWS = MROWS + 8
DSTR = DROWS * 128


SMB = 4


def _smooth_body(d_ref, p_ref, o_ref):
    for i in range(SMB):
        a = d_ref[i, 0]
        zr = jnp.zeros((1, W), jnp.float32)
        rs = a + jnp.concatenate([a[1:], zr], 0) + jnp.concatenate([zr, a[:-1]], 0)
        zc = jnp.zeros((H, 1), jnp.float32)
        cs = rs + jnp.concatenate([rs[:, 1:], zc], 1) + jnp.concatenate([zc, rs[:, :-1]], 1)
        o_ref[i, :MROWS] = (cs * jnp.float32(1.0 / 9.0)).reshape(MROWS, 128)
        ybits = jax.lax.bitcast_convert_type(p_ref[i, :, 0], jnp.float32)
        xbits = jax.lax.bitcast_convert_type(p_ref[i, :, 1], jnp.float32)
        o_ref[i, MROWS:MROWS + 2] = ybits.reshape(2, 128)
        o_ref[i, MROWS + 2:MROWS + 4] = xbits.reshape(2, 128)
        o_ref[i, MROWS + 4:] = jnp.zeros((4, 128), jnp.float32)


_smooth_call = pl.pallas_call(
    _smooth_body,
    out_shape=jax.ShapeDtypeStruct((B, DROWS, 128), jnp.float32),
    grid=(B // SMB,),
    in_specs=[pl.BlockSpec((SMB, 1, H, W), lambda b: (b, 0, 0, 0)),
              pl.BlockSpec((SMB, N, 2), lambda b: (b, 0, 0))],
    out_specs=pl.BlockSpec((SMB, DROWS, 128), lambda b: (b, 0, 0)),
)


def _climb_body(d_hbm, out_hbm,
                pin, ycur, xcur, cv, idxA, valsA, idxB, valsB, obuf, shared,
                semA, semB):
    cid = lax.axis_index("c")
    sid = lax.axis_index("s")
    wid = cid * NS + sid
    base_pt = wid * PPW
    bat = cid * WPB + sid // WPB
    quarter = sid % WPB
    boff = (sid // WPB) * (H * W)

    SEG = H * W // WPB
    pltpu.sync_copy(d_hbm.at[pl.ds(bat * DSTR + quarter * SEG, SEG)],
                    shared.at[pl.ds(sid * SEG, SEG)])

    lane16 = lax.iota(jnp.int32, L)

    pltpu.sync_copy(d_hbm.at[pl.ds(bat * DSTR + H * W + quarter * PPW, PPW)],
                    pin.at[pl.ds(0, PPW)])
    pltpu.sync_copy(d_hbm.at[pl.ds(bat * DSTR + H * W + N + quarter * PPW, PPW)],
                    pin.at[pl.ds(PPW, PPW)])
    plsc.subcore_barrier()
    for g in range(GRPS):
        sl = pl.ds(g * L, L)
        ycur[sl] = plsc.bitcast(pin[pl.ds(g * L, L)], jnp.int32)
        xcur[sl] = plsc.bitcast(pin[pl.ds(PPW + g * L, L)], jnp.int32)

    halves = ((idxA, valsA, semA, 0), (idxB, valsB, semB, 1))

    def compute_idx(idx_ref, h):
        for g in range(HG):
            g_abs = 2 * h + g
            sl = pl.ds(g_abs * L, L)
            yv = ycur[sl]
            xv = xcur[sl]
            cyd = {dy: (jnp.clip(yv + dy, 0, H - 1) << 9) + boff for dy in (-1, 0, 1)}
            cxd = {dx: jnp.clip(xv + dx, 0, W - 1) for dx in (-1, 0, 1)}
            for k, (dy, dx) in enumerate(OFFS8):
                idx_ref[pl.ds(k * HPTS + g * L, L)] = cyd[dy] + cxd[dx]

    def fire(idx_ref, vals_ref, sem):
        return [pltpu.async_copy(shared.at[idx_ref.at[pl.ds(o, 128)]],
                                 vals_ref.at[pl.ds(o, 128)], sem)
                for o in (0, 128)]

    def drain(copies):
        for c in copies:
            c.wait()

    def advance(vals_ref, h):
        for g in range(HG):
            g_abs = 2 * h + g
            sl = pl.ds(g_abs * L, L)
            yv = ycur[sl]
            xv = xcur[sl]
            bv = bdy = bdx = None
            k8 = 0
            for (dy, dx) in OFFS:
                if (dy, dx) == (0, 0):
                    val = cv[sl]
                else:
                    val = vals_ref[pl.ds(k8 * HPTS + g * L, L)]
                    k8 += 1
                if bv is None:
                    bv = val
                    bdy = jnp.full((L,), dy, jnp.int32)
                    bdx = jnp.full((L,), dx, jnp.int32)
                else:
                    m = val > bv
                    bv = jnp.where(m, val, bv)
                    bdy = jnp.where(m, jnp.int32(dy), bdy)
                    bdx = jnp.where(m, jnp.int32(dx), bdx)
            cv[sl] = bv
            ycur[sl] = jnp.clip(yv + bdy, 0, H - 1)
            xcur[sl] = jnp.clip(xv + bdx, 0, W - 1)

    for g in range(GRPS):
        sl = pl.ds(g * L, L)
        idxA[sl] = boff + (ycur[sl] << 9) + xcur[sl]
    pltpu.async_copy(shared.at[idxA.at[pl.ds(0, PPW)]], cv, semA).wait()

    compute_idx(idxA, 0)
    fire(idxA, valsA, semA)
    compute_idx(idxB, 1)
    fire(idxB, valsB, semB)

    def body2(_, carry):
        drain([pltpu.make_async_copy(shared.at[idxA.at[pl.ds(o, 128)]],
                                     valsA.at[pl.ds(o, 128)], semA)
               for o in (0, 128)])
        advance(valsA, 0)
        compute_idx(idxA, 0)
        fire(idxA, valsA, semA)
        drain([pltpu.make_async_copy(shared.at[idxB.at[pl.ds(o, 128)]],
                                     valsB.at[pl.ds(o, 128)], semB)
               for o in (0, 128)])
        advance(valsB, 1)
        compute_idx(idxB, 1)
        fire(idxB, valsB, semB)
        return carry

    lax.fori_loop(0, MAX_ITERS, body2, 0)

    drain([pltpu.make_async_copy(shared.at[idxA.at[pl.ds(o, 128)]],
                                 valsA.at[pl.ds(o, 128)], semA)
           for o in (0, 128)])
    drain([pltpu.make_async_copy(shared.at[idxB.at[pl.ds(o, 128)]],
                                 valsB.at[pl.ds(o, 128)], semB)
           for o in (0, 128)])

    for g in range(GRPS):
        sl = pl.ds(g * L, L)
        oi = (lane16 << 1) + (2 * g * L)
        plsc.store_scatter(obuf, [oi], ycur[sl].astype(jnp.float32))
        plsc.store_scatter(obuf, [oi + 1], xcur[sl].astype(jnp.float32))
    pltpu.sync_copy(obuf, out_hbm.at[pl.ds(base_pt * 2, 2 * PPW)])


@functools.cache
def _climb_call():
    return pl.kernel(
        _climb_body,
        out_type=jax.ShapeDtypeStruct((PTS * 2,), jnp.float32),
        mesh=plsc.VectorSubcoreMesh(core_axis_name="c", subcore_axis_name="s",
                                    num_cores=NC, num_subcores=NS),
        scratch_types=[
            pltpu.VMEM((2 * PPW,), jnp.float32),
            pltpu.VMEM((PPW,), jnp.int32),
            pltpu.VMEM((PPW,), jnp.int32),
            pltpu.VMEM((PPW,), jnp.float32),
            pltpu.VMEM((HFLAT,), jnp.int32),
            pltpu.VMEM((HFLAT,), jnp.float32),
            pltpu.VMEM((HFLAT,), jnp.int32),
            pltpu.VMEM((HFLAT,), jnp.float32),
            pltpu.VMEM((2 * PPW,), jnp.float32),
            pltpu.VMEM_SHARED((WPB * H * W,), jnp.float32),
            pltpu.SemaphoreType.DMA,
            pltpu.SemaphoreType.DMA,
        ],
        compiler_params=pltpu.CompilerParams(needs_layout_passes=False),
    )


def kernel(depth, points):
    d = _smooth_call(depth, points)
    out = _climb_call()(d.reshape(B * DSTR))
    return out.reshape(B, N, 2)
